# baseline (speedup 1.0000x reference)
import jax
import jax.numpy as jnp
from jax.experimental import pallas as pl
from jax.experimental.pallas import tpu as pltpu

N_CODES = 1024
EDIM = 64
BETA = 0.25


def _quantize_one(z_n, emb, emb2, zpsq, embsq):
    T = z_n.shape[1]
    K = emb.shape[0]

    zp = z_n.T

    dot2 = jax.lax.dot_general(zp, emb2, (((1,), (1,)), ((), ())))
    d = (zpsq + embsq) - dot2

    dmin = jnp.min(d, axis=1, keepdims=True)
    iota_k = jax.lax.broadcasted_iota(jnp.int32, (T, K), 1)
    idx = jnp.min(jnp.where(d == dmin, iota_k, K), axis=1)

    p = (iota_k == idx[:, None]).astype(jnp.float32)
    zqt = jax.lax.dot_general(emb, p, (((0,), (1,)), ((), ())))

    part = jnp.sum(dmin)
    ones_t = jnp.ones((1, T), jnp.float32)
    cnt = jax.lax.dot_general(ones_t, p, (((1,), (0,)), ((), ())))

    return z_n + (zqt - z_n), part, cnt


def _vq_kernel(z_ref, emb_ref, emb2_ref, zpsq_ref, embsq_ref,
               zq_ref, loss_ref, perp_ref, counts_ref, acc_ref):
    i = pl.program_id(0)
    nsteps = pl.num_programs(0)
    emb = emb_ref[...]
    emb2 = emb2_ref[...]
    embsq = embsq_ref[...]
    nb = z_ref.shape[0]
    T = z_ref.shape[2]

    parts = []
    cnts = []
    for b in range(nb):
        out_b, part_b, cnt_b = _quantize_one(
            z_ref[b], emb, emb2,
            zpsq_ref[pl.ds(b * T, T), :], embsq)
        zq_ref[b] = out_b
        parts.append(part_b)
        cnts.append(cnt_b)
    part = parts[0] + parts[1] if nb == 2 else sum(parts)
    cnt = cnts[0] + cnts[1] if nb == 2 else sum(cnts)

    @pl.when(i == 0)
    def _():
        acc_ref[0, 0] = 0.0
        counts_ref[...] = jnp.zeros_like(counts_ref)

    acc_ref[0, 0] += part
    counts_ref[...] += cnt

    @pl.when(i == nsteps - 1)
    def _():
        total_rows = nsteps * nb * T
        m = acc_ref[0, 0] / (total_rows * EDIM)
        loss_ref[...] = jnp.reshape(m + BETA * m, (1, 1))
        e_mean = counts_ref[...] / total_rows
        plogp = e_mean * jnp.log(e_mean + 1e-10)
        perp_ref[...] = jnp.reshape(jnp.exp(-jnp.sum(plogp)), (1, 1))


def kernel(z, emb):
    N, W, T = z.shape
    K = emb.shape[0]
    zpsq = jnp.sum(z ** 2, axis=1).reshape(-1, 1)
    embsq = jnp.sum(emb ** 2, axis=1)[None, :]
    emb2 = emb + emb
    NB = 2
    zq, loss, perp = pl.pallas_call(
        _vq_kernel,
        grid=(N // NB,),
        in_specs=[
            pl.BlockSpec((NB, W, T), lambda i: (i, 0, 0)),
            pl.BlockSpec((K, W), lambda i: (0, 0)),
            pl.BlockSpec((K, W), lambda i: (0, 0)),
            pl.BlockSpec((NB * T, 1), lambda i: (i, 0)),
            pl.BlockSpec((1, K), lambda i: (0, 0)),
        ],
        out_specs=[
            pl.BlockSpec((NB, W, T), lambda i: (i, 0, 0)),
            pl.BlockSpec((1, 1), lambda i: (0, 0)),
            pl.BlockSpec((1, 1), lambda i: (0, 0)),
        ],
        out_shape=[
            jax.ShapeDtypeStruct((N, W, T), jnp.float32),
            jax.ShapeDtypeStruct((1, 1), jnp.float32),
            jax.ShapeDtypeStruct((1, 1), jnp.float32),
        ],
        scratch_shapes=[
            pltpu.VMEM((1, K), jnp.float32),
            pltpu.SMEM((1, 1), jnp.float32),
        ],
        compiler_params=pltpu.CompilerParams(
            dimension_semantics=("arbitrary",)),
    )(z, emb, emb2, zpsq, embsq)
    return zq, loss[0, 0], perp[0, 0]


# speedup vs baseline: 1.0416x; 1.0416x over previous
import jax
import jax.numpy as jnp
from jax.experimental import pallas as pl
from jax.experimental.pallas import tpu as pltpu

N_CODES = 1024
EDIM = 64
BETA = 0.25


def _quantize_one(z_n, emb, emb2, zpsq, embsq):
    T = z_n.shape[1]
    K = emb.shape[0]

    zp = z_n.T

    dot2 = jax.lax.dot_general(zp, emb2, (((1,), (1,)), ((), ())))
    d = (zpsq + embsq) - dot2

    dmin = jnp.min(d, axis=1, keepdims=True)
    iota_k = jax.lax.broadcasted_iota(jnp.int32, (T, K), 1)
    idx = jnp.min(jnp.where(d == dmin, iota_k, K), axis=1)

    p = (iota_k == idx[:, None]).astype(jnp.float32)
    zqt = jax.lax.dot_general(emb, p, (((0,), (1,)), ((), ())))

    part = jnp.sum(dmin)
    ones_t = jnp.ones((1, T), jnp.float32)
    cnt = jax.lax.dot_general(ones_t, p, (((1,), (0,)), ((), ())))

    return z_n + (zqt - z_n), part, cnt


def _vq_kernel(z_ref, emb_ref, emb2_ref, zpsq_ref, embsq_ref,
               zq_ref, loss_ref, perp_ref, counts_ref, acc_ref):
    i = pl.program_id(0)
    nsteps = pl.num_programs(0)
    emb = emb_ref[...]
    emb2 = emb2_ref[...]
    embsq = embsq_ref[...]
    nb = z_ref.shape[0]
    T = z_ref.shape[2]

    parts = []
    cnts = []
    for b in range(nb):
        out_b, part_b, cnt_b = _quantize_one(
            z_ref[b], emb, emb2,
            zpsq_ref[pl.ds(b * T, T), :], embsq)
        zq_ref[b] = out_b
        parts.append(part_b)
        cnts.append(cnt_b)
    part = parts[0] + parts[1] if nb == 2 else sum(parts)
    cnt = cnts[0] + cnts[1] if nb == 2 else sum(cnts)

    @pl.when(i == 0)
    def _():
        acc_ref[0, 0] = 0.0
        counts_ref[...] = jnp.zeros_like(counts_ref)

    acc_ref[0, 0] += part
    counts_ref[...] += cnt

    @pl.when(i == nsteps - 1)
    def _():
        total_rows = nsteps * nb * T
        m = acc_ref[0, 0] / (total_rows * EDIM)
        loss_ref[...] = jnp.reshape(m + BETA * m, (1, 1))
        e_mean = counts_ref[...] / total_rows
        plogp = e_mean * jnp.log(e_mean + 1e-10)
        perp_ref[...] = jnp.reshape(jnp.exp(-jnp.sum(plogp)), (1, 1))


def kernel(z, emb):
    N, W, T = z.shape
    K = emb.shape[0]
    zpsq = jnp.sum(z ** 2, axis=1).reshape(-1, 1)
    embsq = jnp.sum(emb ** 2, axis=1)[None, :]
    emb2 = emb + emb
    NB = 1
    zq, loss, perp = pl.pallas_call(
        _vq_kernel,
        grid=(N // NB,),
        in_specs=[
            pl.BlockSpec((NB, W, T), lambda i: (i, 0, 0)),
            pl.BlockSpec((K, W), lambda i: (0, 0)),
            pl.BlockSpec((K, W), lambda i: (0, 0)),
            pl.BlockSpec((NB * T, 1), lambda i: (i, 0)),
            pl.BlockSpec((1, K), lambda i: (0, 0)),
        ],
        out_specs=[
            pl.BlockSpec((NB, W, T), lambda i: (i, 0, 0)),
            pl.BlockSpec((1, 1), lambda i: (0, 0)),
            pl.BlockSpec((1, 1), lambda i: (0, 0)),
        ],
        out_shape=[
            jax.ShapeDtypeStruct((N, W, T), jnp.float32),
            jax.ShapeDtypeStruct((1, 1), jnp.float32),
            jax.ShapeDtypeStruct((1, 1), jnp.float32),
        ],
        scratch_shapes=[
            pltpu.VMEM((1, K), jnp.float32),
            pltpu.SMEM((1, 1), jnp.float32),
        ],
        compiler_params=pltpu.CompilerParams(
            dimension_semantics=("arbitrary",)),
    )(z, emb, emb2, zpsq, embsq)
    return zq, loss[0, 0], perp[0, 0]
